# Initial kernel scaffold; baseline (speedup 1.0000x reference)
import jax
import jax.numpy as jnp
from jax.experimental import pallas as pl


def kernel(adv_pcs, ori_pcs):
    raise NotImplementedError("write your pallas kernel here")


# speedup vs baseline: 21.5372x; 21.5372x over previous
import jax
import jax.numpy as jnp
from jax.experimental import pallas as pl

B = 8
N = 2048
KSEL = 17
RB = 256


def _body(ori_r, adv_r, oriT_r, advT_r, out_r):
    rows_o = ori_r[0]
    rows_a = adv_r[0]
    colsT_o = oriT_r[0]
    colsT_a = advT_r[0]

    sq_r = jnp.sum(rows_o * rows_o, axis=1, keepdims=True)
    sq_c = jnp.sum(colsT_o * colsT_o, axis=0, keepdims=True)
    cross = jax.lax.dot_general(
        rows_o, colsT_o, (((1,), (0,)), ((), ())),
        preferred_element_type=jnp.float32)
    dist = jnp.maximum(sq_r - 2.0 * cross + sq_c, 0.0)

    colidx = jax.lax.broadcasted_iota(jnp.int32, (RB, N), 1)
    packed = (jax.lax.bitcast_convert_type(dist, jnp.int32)
              & jnp.int32(-2048)) | colidx

    work = packed
    thresh = None
    for _ in range(KSEL):
        thresh = jnp.min(work, axis=1, keepdims=True)
        work = jnp.where(work == thresh, jnp.int32(0x7FFFFFFF), work)
    mask = (packed <= thresh).astype(jnp.float32)

    dispT = colsT_a - colsT_o
    qT = jnp.sum(dispT * dispT, axis=0, keepdims=True)
    s1 = jnp.sum(mask * qT, axis=1, keepdims=True)
    v = jax.lax.dot_general(
        mask, dispT, (((1,), (1,)), ((), ())),
        preferred_element_type=jnp.float32)

    disp_rows = rows_a - rows_o
    q_rows = jnp.sum(disp_rows * disp_rows, axis=1, keepdims=True)
    dotrv = jnp.sum(disp_rows * v, axis=1, keepdims=True)
    contrib = float(KSEL) * q_rows + s1 - 2.0 * dotrv
    out_r[...] = jnp.sum(contrib).reshape(1, 1, 1)


def kernel(adv_pcs, ori_pcs):
    oriT = ori_pcs.transpose(0, 2, 1)
    advT = adv_pcs.transpose(0, 2, 1)
    nrb = N // RB
    partials = pl.pallas_call(
        _body,
        grid=(B, nrb),
        in_specs=[
            pl.BlockSpec((1, RB, 3), lambda b, r: (b, r, 0)),
            pl.BlockSpec((1, RB, 3), lambda b, r: (b, r, 0)),
            pl.BlockSpec((1, 3, N), lambda b, r: (b, 0, 0)),
            pl.BlockSpec((1, 3, N), lambda b, r: (b, 0, 0)),
        ],
        out_specs=pl.BlockSpec((1, 1, 1), lambda b, r: (b * nrb + r, 0, 0)),
        out_shape=jax.ShapeDtypeStruct((B * nrb, 1, 1), jnp.float32),
    )(ori_pcs, adv_pcs, oriT, advT)
    return jnp.sum(partials) / jnp.float32(B * N * (KSEL - 1))


# speedup vs baseline: 30.0620x; 1.3958x over previous
import jax
import jax.numpy as jnp
from jax.experimental import pallas as pl

B = 8
N = 2048
KSEL = 17
RB = 256


def _body(ori_r, adv_r, oriT_r, advT_r, out_r):
    rows_o = ori_r[0]
    rows_a = adv_r[0]
    colsT_o = oriT_r[0]
    colsT_a = advT_r[0]

    sq_r = jnp.sum(rows_o * rows_o, axis=1, keepdims=True)
    sq_c = jnp.sum(colsT_o * colsT_o, axis=0, keepdims=True)
    cross = jax.lax.dot_general(
        rows_o, colsT_o, (((1,), (0,)), ((), ())),
        preferred_element_type=jnp.float32)
    dist = jnp.maximum(sq_r - 2.0 * cross + sq_c, 0.0)

    colidx = jax.lax.broadcasted_iota(jnp.int32, (RB, N), 1)
    packed = (jax.lax.bitcast_convert_type(dist, jnp.int32)
              & jnp.int32(-2048)) | colidx

    NKEEP = 5
    imax = jnp.int32(0x7FFFFFFF)
    d_lvls = [jnp.full((RB, N // 16), imax, jnp.int32) for _ in range(NKEEP)]
    for c in range(16):
        x = packed[:, c * (N // 16):(c + 1) * (N // 16)]
        for l in range(NKEEP - 1):
            lo = jnp.minimum(d_lvls[l], x)
            x = jnp.maximum(d_lvls[l], x)
            d_lvls[l] = lo
        d_lvls[NKEEP - 1] = jnp.minimum(d_lvls[NKEEP - 1], x)

    thresh = None
    for it in range(KSEL):
        thresh = jnp.min(d_lvls[0], axis=1, keepdims=True)
        if it < KSEL - 1:
            one = d_lvls[0] == thresh
            for l in range(NKEEP - 1):
                d_lvls[l] = jnp.where(one, d_lvls[l + 1], d_lvls[l])
            d_lvls[NKEEP - 1] = jnp.where(one, imax, d_lvls[NKEEP - 1])
    mask = (packed <= thresh).astype(jnp.float32)
    count = jnp.sum(mask, axis=1, keepdims=True)

    dispT = colsT_a - colsT_o
    qT = jnp.sum(dispT * dispT, axis=0, keepdims=True)
    s1 = jnp.sum(mask * qT, axis=1, keepdims=True)
    v = jax.lax.dot_general(
        mask, dispT, (((1,), (1,)), ((), ())),
        preferred_element_type=jnp.float32)

    disp_rows = rows_a - rows_o
    q_rows = jnp.sum(disp_rows * disp_rows, axis=1, keepdims=True)
    dotrv = jnp.sum(disp_rows * v, axis=1, keepdims=True)
    contrib = count * q_rows + s1 - 2.0 * dotrv
    out_r[...] = jnp.sum(contrib).reshape(1, 1, 1)


def kernel(adv_pcs, ori_pcs):
    oriT = ori_pcs.transpose(0, 2, 1)
    advT = adv_pcs.transpose(0, 2, 1)
    nrb = N // RB
    partials = pl.pallas_call(
        _body,
        grid=(B, nrb),
        in_specs=[
            pl.BlockSpec((1, RB, 3), lambda b, r: (b, r, 0)),
            pl.BlockSpec((1, RB, 3), lambda b, r: (b, r, 0)),
            pl.BlockSpec((1, 3, N), lambda b, r: (b, 0, 0)),
            pl.BlockSpec((1, 3, N), lambda b, r: (b, 0, 0)),
        ],
        out_specs=pl.BlockSpec((1, 1, 1), lambda b, r: (b * nrb + r, 0, 0)),
        out_shape=jax.ShapeDtypeStruct((B * nrb, 1, 1), jnp.float32),
    )(ori_pcs, adv_pcs, oriT, advT)
    return jnp.sum(partials) / jnp.float32(B * N * (KSEL - 1))
